# baseline (speedup 1.0000x reference)
import functools

import jax
import jax.numpy as jnp
from jax import lax
from jax.experimental import pallas as pl
from jax.experimental.pallas import tpu as pltpu
from jax.experimental.pallas import tpu_sc as plsc

B = 16384
LP = 20
LN = 200
D = 64
TPAD = 224
CHUNK = 112
W = 32
RPW = B // W
RB = 64
NBLK = RPW // RB

_NC = 2
_NSC = 16


def _sc_logits(tgt3, urows_all, vrows_all, combo):
  mesh = plsc.VectorSubcoreMesh(
      core_axis_name="c", subcore_axis_name="s", num_cores=_NC,
      num_subcores=_NSC)

  @functools.partial(
      pl.kernel,
      mesh=mesh,
      compiler_params=pltpu.CompilerParams(
          needs_layout_passes=False, use_tc_tiling_on_sc=False),
      out_type=jax.ShapeDtypeStruct((B, TPAD), jnp.float32),
      scratch_types=[
          pltpu.VMEM((RB, 2, CHUNK), jnp.int32),
          pltpu.VMEM((RB, D), jnp.float32),
          pltpu.VMEM((RB, D), jnp.float32),
          pltpu.VMEM((4, 2, CHUNK, 2 * D), jnp.bfloat16),
          pltpu.VMEM((RB, TPAD), jnp.float32),
          pltpu.VMEM((16, 16), jnp.float32),
          pltpu.SemaphoreType.DMA,
          pltpu.SemaphoreType.DMA,
          pltpu.SemaphoreType.DMA,
          pltpu.SemaphoreType.DMA,
          pltpu.SemaphoreType.DMA,
      ],
  )
  def k(tgt_hbm, ur_hbm, vr_hbm, combo_hbm,
        out_hbm, tidx, urows, vrows, tu, lbuf, strans, sem,
        sem0, sem1, sem2, sem3):
    wid = lax.axis_index("s") * _NC + lax.axis_index("c")

    def do_block(blk, _):
      base = wid * RPW + blk * RB
      pltpu.sync_copy(tgt_hbm.at[pl.ds(base, RB)], tidx)
      pltpu.sync_copy(ur_hbm.at[pl.ds(base, RB)], urows)
      pltpu.sync_copy(vr_hbm.at[pl.ds(base, RB)], vrows)

      iota16 = lax.broadcasted_iota(jnp.int32, (16,), 0)
      ngrp = TPAD // 16
      sems = [sem0, sem1, sem2, sem3]

      def fire(r, slot):
        sm = sems[slot]
        return [
            pltpu.async_copy(combo_hbm.at[tidx.at[r, 0]], tu.at[slot, 0], sm),
            pltpu.async_copy(combo_hbm.at[tidx.at[r, 1]], tu.at[slot, 1], sm),
        ]

      def drain(slot):
        for cp in fire_dummy[slot]:
          cp.wait()

      def compute(r, slot):
        rr = jnp.full((16,), r, dtype=jnp.int32)
        ev = iota16 * 2
        od = ev + 1
        uve = ([plsc.load_gather(urows, [rr, ev + 32 * j]) for j in range(2)]
               + [plsc.load_gather(vrows, [rr, ev + 32 * j]) for j in range(2)])
        uvo = ([plsc.load_gather(urows, [rr, od + 32 * j]) for j in range(2)]
               + [plsc.load_gather(vrows, [rr, od + 32 * j]) for j in range(2)])
        def do_grp(g, _):
          c = g // 7
          tus = tu.at[slot, c]
          base_t = g * 16
          for l in range(16):
            t = base_t + l - c * CHUNK
            ps = []
            for j in range(4):
              eu, ou = plsc.unpack(tus[t, pl.ds(32 * j, 32)],
                                   format=plsc.PackFormat.INTERLEAVED)
              ps.append(eu * uve[j] + ou * uvo[j])
            strans[l, pl.ds(0, 16)] = (ps[0] + ps[1]) + (ps[2] + ps[3])
          cols = [plsc.load_gather(
              strans, [iota16, jnp.full((16,), cc, dtype=jnp.int32)])
              for cc in range(16)]
          while len(cols) > 1:
            cols = [a + b for a, b in zip(cols[::2], cols[1::2])]
          lbuf[r, pl.ds(g * 16, 16)] = cols[0]
          return 0

        lax.fori_loop(0, ngrp, do_grp, 0, unroll=2)

      fire_dummy = [fire(0, 0), fire(1, 1), fire(2, 2), fire(3, 3)]

      def do_quad(rq, _):
        r0 = 4 * rq
        for sl in range(4):
          drain(sl)
          compute(r0 + sl, sl)

          @pl.when(r0 + sl + 4 < RB)
          def _(sl=sl):
            fire(r0 + sl + 4, sl)

        return 0

      lax.fori_loop(0, RB // 4, do_quad, 0)
      pltpu.sync_copy(lbuf, out_hbm.at[pl.ds(base, RB)])
      return 0

    lax.fori_loop(0, NBLK, do_block, 0)

  return k(tgt3, urows_all, vrows_all, combo)


MB = 512
MW = 256


def _mask_body(pl_ref, nl_ref, mask_ref, nt_ref):
  col = lax.broadcasted_iota(jnp.int32, (MB, MW), 1)
  p = pl_ref[...]
  n = nl_ref[...]
  a = jnp.clip(p - col, 0, 1)
  b = jnp.clip(n - (col - LP), 0, 1)
  ip = jnp.clip(LP - col, 0, 1)
  mask_ref[...] = ip * a + (1 - ip) * b
  nt_ref[...] = ip * a


def _masks(pos_lens, neg_lens):
  pl2 = pos_lens.reshape(B, 1).astype(jnp.int32)
  nl2 = neg_lens.reshape(B, 1).astype(jnp.int32)
  return pl.pallas_call(
      _mask_body,
      grid=(B // MB,),
      in_specs=[pl.BlockSpec((MB, 1), lambda i: (i, 0)),
                pl.BlockSpec((MB, 1), lambda i: (i, 0))],
      out_specs=[pl.BlockSpec((MB, MW), lambda i: (i, 0)),
                 pl.BlockSpec((MB, MW), lambda i: (i, 0))],
      out_shape=[jax.ShapeDtypeStruct((B, MW), jnp.int32),
                 jax.ShapeDtypeStruct((B, MW), jnp.int32)],
  )(pl2, nl2)


def kernel(attr_item, attr_tf_item, attr_lens_item, item_ids, attr_user,
           attr_tf_user, attr_lens_user, user_ids, pos_targets, pos_lens,
           neg_targets, neg_lens, user_table, item_table,
           out_attr_user_table, out_attr_item_table):
  tgt = jnp.concatenate(
      [pos_targets.astype(jnp.int32), neg_targets.astype(jnp.int32),
       jnp.zeros((B, TPAD - LP - LN), jnp.int32)], axis=1).reshape(B, 2, CHUNK)
  urows_all = jnp.take(user_table, user_ids, axis=0)
  vrows_all = jnp.take(item_table, item_ids, axis=0)
  combo = jnp.concatenate([out_attr_user_table.astype(jnp.bfloat16),
                           out_attr_item_table.astype(jnp.bfloat16)], axis=1)
  logits_pad = _sc_logits(tgt, urows_all, vrows_all, combo)
  logits = logits_pad[:, :LP + LN]
  mask_pad, nt_pad = _masks(pos_lens, neg_lens)
  return (logits, mask_pad[:, :LP + LN] != 0, nt_pad[:, :LP + LN])


# speedup vs baseline: 1.0136x; 1.0136x over previous
import functools

import jax
import jax.numpy as jnp
from jax import lax
from jax.experimental import pallas as pl
from jax.experimental.pallas import tpu as pltpu
from jax.experimental.pallas import tpu_sc as plsc

B = 16384
LP = 20
LN = 200
D = 64
TPAD = 224
CHUNK = 112
W = 32
RPW = B // W
RB = 64
NBLK = RPW // RB

_NC = 2
_NSC = 16


def _sc_logits(tgt3, urows_all, vrows_all, aut, ait):
  mesh = plsc.VectorSubcoreMesh(
      core_axis_name="c", subcore_axis_name="s", num_cores=_NC,
      num_subcores=_NSC)

  @functools.partial(
      pl.kernel,
      mesh=mesh,
      compiler_params=pltpu.CompilerParams(
          needs_layout_passes=False, use_tc_tiling_on_sc=False),
      out_type=jax.ShapeDtypeStruct((B, TPAD), jnp.float32),
      scratch_types=[
          pltpu.VMEM((RB, 2, CHUNK), jnp.int32),
          pltpu.VMEM((RB, D), jnp.float32),
          pltpu.VMEM((RB, D), jnp.float32),
          pltpu.VMEM((4, 2, CHUNK, D), jnp.bfloat16),
          pltpu.VMEM((4, 2, CHUNK, D), jnp.bfloat16),
          pltpu.VMEM((RB, TPAD), jnp.float32),
          pltpu.VMEM((16, 16), jnp.float32),
          pltpu.SemaphoreType.DMA,
          pltpu.SemaphoreType.DMA,
          pltpu.SemaphoreType.DMA,
          pltpu.SemaphoreType.DMA,
          pltpu.SemaphoreType.DMA,
      ],
  )
  def k(tgt_hbm, ur_hbm, vr_hbm, aut_hbm, ait_hbm,
        out_hbm, tidx, urows, vrows, tu, ti, lbuf, strans, sem,
        sem0, sem1, sem2, sem3):
    wid = lax.axis_index("s") * _NC + lax.axis_index("c")

    def do_block(blk, _):
      base = wid * RPW + blk * RB
      pltpu.sync_copy(tgt_hbm.at[pl.ds(base, RB)], tidx)
      pltpu.sync_copy(ur_hbm.at[pl.ds(base, RB)], urows)
      pltpu.sync_copy(vr_hbm.at[pl.ds(base, RB)], vrows)

      iota16 = lax.broadcasted_iota(jnp.int32, (16,), 0)
      ngrp = TPAD // 16
      sems = [sem0, sem1, sem2, sem3]

      def fire(r, slot):
        sm = sems[slot]
        return [
            pltpu.async_copy(aut_hbm.at[tidx.at[r, 0]], tu.at[slot, 0], sm),
            pltpu.async_copy(aut_hbm.at[tidx.at[r, 1]], tu.at[slot, 1], sm),
            pltpu.async_copy(ait_hbm.at[tidx.at[r, 0]], ti.at[slot, 0], sm),
            pltpu.async_copy(ait_hbm.at[tidx.at[r, 1]], ti.at[slot, 1], sm),
        ]

      def drain(slot):
        for cp in fire_dummy[slot]:
          cp.wait()

      def compute(r, slot):
        rr = jnp.full((16,), r, dtype=jnp.int32)
        ev = iota16 * 2
        od = ev + 1
        ue = [plsc.load_gather(urows, [rr, ev + 32 * j]) for j in range(2)]
        uo = [plsc.load_gather(urows, [rr, od + 32 * j]) for j in range(2)]
        ve = [plsc.load_gather(vrows, [rr, ev + 32 * j]) for j in range(2)]
        vo = [plsc.load_gather(vrows, [rr, od + 32 * j]) for j in range(2)]
        def do_grp(g, _):
          c = g // 7
          tus = tu.at[slot, c]
          tis = ti.at[slot, c]
          base_t = g * 16
          for l in range(16):
            t = base_t + l - c * CHUNK
            ps = []
            for j in range(2):
              eu, ou = plsc.unpack(tus[t, pl.ds(32 * j, 32)],
                                   format=plsc.PackFormat.INTERLEAVED)
              ei, oi = plsc.unpack(tis[t, pl.ds(32 * j, 32)],
                                   format=plsc.PackFormat.INTERLEAVED)
              ps.append(eu * ue[j] + ou * uo[j])
              ps.append(ei * ve[j] + oi * vo[j])
            strans[l, pl.ds(0, 16)] = (ps[0] + ps[1]) + (ps[2] + ps[3])
          cols = [plsc.load_gather(
              strans, [iota16, jnp.full((16,), cc, dtype=jnp.int32)])
              for cc in range(16)]
          while len(cols) > 1:
            cols = [a + b for a, b in zip(cols[::2], cols[1::2])]
          lbuf[r, pl.ds(g * 16, 16)] = cols[0]
          return 0

        lax.fori_loop(0, ngrp, do_grp, 0, unroll=2)

      fire_dummy = [fire(0, 0), fire(1, 1), fire(2, 2), fire(3, 3)]

      def do_quad(rq, _):
        r0 = 4 * rq
        for sl in range(4):
          drain(sl)
          compute(r0 + sl, sl)

          @pl.when(r0 + sl + 4 < RB)
          def _(sl=sl):
            fire(r0 + sl + 4, sl)

        return 0

      lax.fori_loop(0, RB // 4, do_quad, 0)
      pltpu.sync_copy(lbuf, out_hbm.at[pl.ds(base, RB)])
      return 0

    lax.fori_loop(0, NBLK, do_block, 0)

  return k(tgt3, urows_all, vrows_all, aut, ait)


MB = 512
MW = 256


def _mask_body(pl_ref, nl_ref, mask_ref, nt_ref):
  col = lax.broadcasted_iota(jnp.int32, (MB, MW), 1)
  p = pl_ref[...]
  n = nl_ref[...]
  a = jnp.clip(p - col, 0, 1)
  b = jnp.clip(n - (col - LP), 0, 1)
  ip = jnp.clip(LP - col, 0, 1)
  mask_ref[...] = ip * a + (1 - ip) * b
  nt_ref[...] = ip * a


def _masks(pos_lens, neg_lens):
  pl2 = pos_lens.reshape(B, 1).astype(jnp.int32)
  nl2 = neg_lens.reshape(B, 1).astype(jnp.int32)
  return pl.pallas_call(
      _mask_body,
      grid=(B // MB,),
      in_specs=[pl.BlockSpec((MB, 1), lambda i: (i, 0)),
                pl.BlockSpec((MB, 1), lambda i: (i, 0))],
      out_specs=[pl.BlockSpec((MB, MW), lambda i: (i, 0)),
                 pl.BlockSpec((MB, MW), lambda i: (i, 0))],
      out_shape=[jax.ShapeDtypeStruct((B, MW), jnp.int32),
                 jax.ShapeDtypeStruct((B, MW), jnp.int32)],
  )(pl2, nl2)


def kernel(attr_item, attr_tf_item, attr_lens_item, item_ids, attr_user,
           attr_tf_user, attr_lens_user, user_ids, pos_targets, pos_lens,
           neg_targets, neg_lens, user_table, item_table,
           out_attr_user_table, out_attr_item_table):
  tgt = jnp.concatenate(
      [pos_targets.astype(jnp.int32), neg_targets.astype(jnp.int32),
       jnp.zeros((B, TPAD - LP - LN), jnp.int32)], axis=1).reshape(B, 2, CHUNK)
  urows_all = jnp.take(user_table, user_ids, axis=0)
  vrows_all = jnp.take(item_table, item_ids, axis=0)
  logits_pad = _sc_logits(tgt, urows_all, vrows_all,
                          out_attr_user_table.astype(jnp.bfloat16),
                          out_attr_item_table.astype(jnp.bfloat16))
  logits = logits_pad[:, :LP + LN]
  mask_pad, nt_pad = _masks(pos_lens, neg_lens)
  return (logits, mask_pad[:, :LP + LN] != 0, nt_pad[:, :LP + LN])
